# baseline (speedup 1.0000x reference)
import functools

import jax
import jax.numpy as jnp
from jax import lax
from jax.experimental import pallas as pl
from jax.experimental.pallas import tpu as pltpu
from jax.experimental.pallas import tpu_sc as plsc

N_NODES = 10000
N_EDGES = 160000
D_FEAT = 256
NUM_HEADS = 8
CH = 125


def _tc_tables(x, W2, b2):
    M = x.shape[0]
    BM = 1000
    H2 = 2 * NUM_HEADS

    def body(x_ref, w_ref, b_ref, t1_ref, t2_ref):
        y = (
            jnp.dot(x_ref[...], w_ref[...], preferred_element_type=jnp.float32)
            + b_ref[...]
        )
        t1_ref[...] = y
        t2_ref[...] = jnp.concatenate(
            [y[:, NUM_HEADS:], y[:, :NUM_HEADS]], axis=1
        )

    return pl.pallas_call(
        body,
        grid=(M // BM,),
        in_specs=[
            pl.BlockSpec((BM, D_FEAT), lambda i: (i, 0)),
            pl.BlockSpec((D_FEAT, H2), lambda i: (0, 0)),
            pl.BlockSpec((1, H2), lambda i: (0, 0)),
        ],
        out_specs=[
            pl.BlockSpec((BM, H2), lambda i: (i, 0)),
            pl.BlockSpec((BM, H2), lambda i: (i, 0)),
        ],
        out_shape=[
            jax.ShapeDtypeStruct((M, H2), jnp.float32),
            jax.ShapeDtypeStruct((M, H2), jnp.float32),
        ],
    )(x, W2, b2)


def _sc_edge_combine(T1, T2, o2, d2):
    info = plsc.get_sparse_core_info()
    NW = info.num_cores * info.num_subcores
    R = o2.shape[0]
    RPW = R // NW
    mesh = plsc.VectorSubcoreMesh(core_axis_name="c", subcore_axis_name="s")

    @functools.partial(
        pl.kernel,
        out_type=jax.ShapeDtypeStruct((R * CH, NUM_HEADS), jnp.float32),
        mesh=mesh,
        compiler_params=pltpu.CompilerParams(use_tc_tiling_on_sc=False),
        scratch_types=[
            pltpu.VMEM((RPW, CH), jnp.int32),
            pltpu.VMEM((RPW, CH), jnp.int32),
            pltpu.VMEM((CH, 16), jnp.float32),
            pltpu.VMEM((CH, 16), jnp.float32),
            pltpu.VMEM((CH, 16), jnp.float32),
            pltpu.VMEM((CH, 16), jnp.float32),
            pltpu.VMEM((CH, 16), jnp.float32),
            pltpu.VMEM((CH, 16), jnp.float32),
            pltpu.SemaphoreType.DMA,
            pltpu.SemaphoreType.DMA,
            pltpu.SemaphoreType.DMA,
            pltpu.SemaphoreType.DMA,
            pltpu.SemaphoreType.DMA,
            pltpu.SemaphoreType.DMA,
        ],
    )
    def k(t1_hbm, t2_hbm, o_hbm, d_hbm, out_hbm, io, idd, a0, a1, b0, b1,
          o0, o1, sa0, sa1, sb0, sb1, so0, so1):
        wid = lax.axis_index("s") * info.num_cores + lax.axis_index("c")
        base_r = wid * RPW
        pltpu.sync_copy(o_hbm.at[pl.ds(base_r, RPW)], io)
        pltpu.sync_copy(d_hbm.at[pl.ds(base_r, RPW)], idd)

        A = [a0, a1]
        B = [b0, b1]
        O = [o0, o1]
        SA = [sa0, sa1]
        SB = [sb0, sb1]
        SO = [so0, so1]

        def issue(j):
            p = j % 2
            ca = pltpu.make_async_copy(t1_hbm.at[io.at[j]], A[p], SA[p])
            ca.start()
            cb = pltpu.make_async_copy(t2_hbm.at[idd.at[j]], B[p], SB[p])
            cb.start()
            return ca, cb

        def compute(p):
            def body(i, _):
                O[p][i, :] = A[p][i, :] + B[p][i, :]
                return 0

            lax.fori_loop(0, CH, body, 0)

        copies = [None] * RPW
        outc = [None] * RPW
        copies[0] = issue(0)
        for j in range(RPW):
            p = j % 2
            if j + 1 < RPW:
                copies[j + 1] = issue(j + 1)
            ca, cb = copies[j]
            ca.wait()
            cb.wait()
            if j >= 2:
                outc[j - 2].wait()
            compute(p)
            oc = pltpu.make_async_copy(
                O[p].at[:, pl.ds(0, NUM_HEADS)],
                out_hbm.at[pl.ds((base_r + j) * CH, CH)],
                SO[p],
            )
            oc.start()
            outc[j] = oc
        outc[RPW - 2].wait()
        outc[RPW - 1].wait()

    return k(T1, T2, o2, d2)


def kernel(x, edge_index, W, b):
    W2 = jnp.concatenate([W[:D_FEAT], W[D_FEAT:]], axis=1)
    b2 = jnp.concatenate([b, jnp.zeros((NUM_HEADS,), jnp.float32)])
    T1, T2 = _tc_tables(x, W2, b2.reshape(1, 2 * NUM_HEADS))

    ei = edge_index.astype(jnp.int32).reshape(2, N_EDGES // CH, CH)
    return _sc_edge_combine(T1, T2, ei[0], ei[1])


# speedup vs baseline: 1.8963x; 1.8963x over previous
import functools

import jax
import jax.numpy as jnp
from jax import lax
from jax.experimental import pallas as pl
from jax.experimental.pallas import tpu as pltpu
from jax.experimental.pallas import tpu_sc as plsc

N_NODES = 10000
N_EDGES = 160000
D_FEAT = 256
NUM_HEADS = 8
CH = 128
NCHUNK = N_EDGES // CH


def _tc_tables(x, W2, b2):
    M = x.shape[0]
    BM = 1000
    H2 = 2 * NUM_HEADS

    def body(x_ref, w_ref, b_ref, t1_ref, t2_ref):
        y = (
            jnp.dot(x_ref[...], w_ref[...], preferred_element_type=jnp.float32)
            + b_ref[...]
        )
        t1_ref[...] = y
        t2_ref[...] = jnp.concatenate(
            [y[:, NUM_HEADS:], y[:, :NUM_HEADS]], axis=1
        )

    return pl.pallas_call(
        body,
        grid=(M // BM,),
        in_specs=[
            pl.BlockSpec((BM, D_FEAT), lambda i: (i, 0)),
            pl.BlockSpec((D_FEAT, H2), lambda i: (0, 0)),
            pl.BlockSpec((1, H2), lambda i: (0, 0)),
        ],
        out_specs=[
            pl.BlockSpec((BM, H2), lambda i: (i, 0)),
            pl.BlockSpec((BM, H2), lambda i: (i, 0)),
        ],
        out_shape=[
            jax.ShapeDtypeStruct((M, H2), jnp.float32),
            jax.ShapeDtypeStruct((M, H2), jnp.float32),
        ],
    )(x, W2, b2)


def _sc_edge_combine(T1, T2, o2, d2):
    info = plsc.get_sparse_core_info()
    NW = info.num_cores * info.num_subcores
    BASE = NCHUNK // NW
    XTRA = NCHUNK - BASE * NW
    mesh = plsc.VectorSubcoreMesh(core_axis_name="c", subcore_axis_name="s")

    @functools.partial(
        pl.kernel,
        out_type=jax.ShapeDtypeStruct((NCHUNK * CH * NUM_HEADS,), jnp.float32),
        mesh=mesh,
        compiler_params=pltpu.CompilerParams(
            use_tc_tiling_on_sc=False, needs_layout_passes=False
        ),
        scratch_types=[
            pltpu.VMEM((BASE + 1, CH), jnp.int32),
            pltpu.VMEM((BASE + 1, CH), jnp.int32),
            pltpu.VMEM((CH, 16), jnp.float32),
            pltpu.VMEM((CH, 16), jnp.float32),
            pltpu.VMEM((CH, 16), jnp.float32),
            pltpu.VMEM((CH, 16), jnp.float32),
            pltpu.VMEM((CH * 16,), jnp.float32),
            pltpu.VMEM((CH * NUM_HEADS,), jnp.float32),
            pltpu.VMEM((CH * NUM_HEADS,), jnp.float32),
            pltpu.SemaphoreType.DMA,
            pltpu.SemaphoreType.DMA,
            pltpu.SemaphoreType.DMA,
            pltpu.SemaphoreType.DMA,
            pltpu.SemaphoreType.DMA,
            pltpu.SemaphoreType.DMA,
        ],
    )
    def k(t1_hbm, t2_hbm, o_hbm, d_hbm, out_hbm, io, idd, a0, a1, b0, b1,
          rows, o0, o1, sa0, sa1, sb0, sb1, so0, so1):
        wid = lax.axis_index("s") * info.num_cores + lax.axis_index("c")
        base_c = wid * BASE
        has_extra = wid < XTRA
        my_n = BASE
        pltpu.sync_copy(o_hbm.at[pl.ds(base_c, BASE)], io.at[pl.ds(0, BASE)])
        pltpu.sync_copy(d_hbm.at[pl.ds(base_c, BASE)], idd.at[pl.ds(0, BASE)])

        @pl.when(has_extra)
        def _():
            xc = NW * BASE + wid
            pltpu.sync_copy(o_hbm.at[pl.ds(xc, 1)], io.at[pl.ds(BASE, 1)])
            pltpu.sync_copy(d_hbm.at[pl.ds(xc, 1)], idd.at[pl.ds(BASE, 1)])

        A = [a0, a1]
        B = [b0, b1]
        O = [o0, o1]
        SA = [sa0, sa1]
        SB = [sb0, sb1]
        SO = [so0, so1]

        iota = lax.iota(jnp.int32, 16)
        gidx0 = iota * 16

        def issue(j):
            p = j % 2
            ca = pltpu.make_async_copy(t1_hbm.at[io.at[j]], A[p], SA[p])
            ca.start()
            cb = pltpu.make_async_copy(t2_hbm.at[idd.at[j]], B[p], SB[p])
            cb.start()
            return ca, cb

        def compute(p):
            def rbody(i, _):
                rows[pl.ds(i * 16, 16)] = A[p][i, :] + B[p][i, :]
                return 0

            lax.fori_loop(0, CH, rbody, 0)

            for h in range(NUM_HEADS):
                def tbody(kk, idx):
                    va = plsc.load_gather(rows, [idx])
                    O[p][pl.ds(h * CH + kk * 16, 16)] = va
                    return idx + 256

                lax.fori_loop(0, CH // 16, tbody, gidx0 + h)

        def write_out(j, chunk_id):
            p = j % 2
            oc = pltpu.make_async_copy(
                O[p],
                out_hbm.at[pl.ds(chunk_id * CH * NUM_HEADS, CH * NUM_HEADS)],
                SO[p],
            )
            oc.start()
            return oc

        def gather_handles(j):
            p = j % 2
            ca = pltpu.make_async_copy(t1_hbm.at[io.at[j]], A[p], SA[p])
            cb = pltpu.make_async_copy(t2_hbm.at[idd.at[j]], B[p], SB[p])
            return ca, cb

        copies = [None] * (my_n + 1)
        outc = [None] * (my_n + 1)
        copies[0] = issue(0)
        for j in range(my_n):
            p = j % 2
            if j + 1 < my_n:
                copies[j + 1] = issue(j + 1)
            else:
                @pl.when(has_extra)
                def _():
                    issue(my_n)
            ca, cb = copies[j]
            ca.wait()
            cb.wait()
            if j >= 2:
                outc[j - 2].wait()
            compute(p)
            outc[j] = write_out(j, base_c + j)

        @pl.when(has_extra)
        def _():
            j = my_n
            p = j % 2
            ca, cb = gather_handles(j)
            ca.wait()
            cb.wait()
            outc[j - 2].wait()
            compute(p)
            oc = write_out(j, NW * BASE + wid)
            outc[j - 1].wait()
            oc.wait()

        @pl.when(jnp.logical_not(has_extra))
        def _():
            outc[my_n - 2].wait()
            outc[my_n - 1].wait()

    return k(T1, T2, o2, d2)


def kernel(x, edge_index, W, b):
    W2 = jnp.concatenate([W[:D_FEAT], W[D_FEAT:]], axis=1)
    b2 = jnp.concatenate([b, jnp.zeros((NUM_HEADS,), jnp.float32)])
    T1, T2 = _tc_tables(x, W2, b2.reshape(1, 2 * NUM_HEADS))

    ei = edge_index.astype(jnp.int32).reshape(2, NCHUNK, CH)
    out_flat = _sc_edge_combine(T1, T2, ei[0], ei[1])
    return (
        out_flat.reshape(NCHUNK, NUM_HEADS, CH)
        .transpose(0, 2, 1)
        .reshape(N_EDGES, NUM_HEADS)
    )


# speedup vs baseline: 2.3867x; 1.2586x over previous
import functools

import jax
import jax.numpy as jnp
from jax import lax
from jax.experimental import pallas as pl
from jax.experimental.pallas import tpu as pltpu
from jax.experimental.pallas import tpu_sc as plsc

N_NODES = 10000
N_EDGES = 160000
D_FEAT = 256
NUM_HEADS = 8
CH = 128
NCHUNK = N_EDGES // CH
OUT_W = CH * NUM_HEADS


def _tc_table(x, W2, b2):
    M = x.shape[0]
    BM = 2000
    H2 = 2 * NUM_HEADS

    def body(x_ref, w_ref, b_ref, t_ref):
        t_ref[...] = (
            jnp.dot(x_ref[...], w_ref[...], preferred_element_type=jnp.float32)
            + b_ref[...]
        )

    return pl.pallas_call(
        body,
        grid=(M // BM,),
        in_specs=[
            pl.BlockSpec((BM, D_FEAT), lambda i: (i, 0)),
            pl.BlockSpec((D_FEAT, H2), lambda i: (0, 0)),
            pl.BlockSpec((1, H2), lambda i: (0, 0)),
        ],
        out_specs=pl.BlockSpec((BM, H2), lambda i: (i, 0)),
        out_shape=jax.ShapeDtypeStruct((M, H2), jnp.float32),
    )(x, W2, b2)


def _sc_edge_combine(T, o2, d2):
    info = plsc.get_sparse_core_info()
    NW = info.num_cores * info.num_subcores
    CPW = 40
    LASTN = NCHUNK - (NW - 1) * CPW
    mesh = plsc.VectorSubcoreMesh(core_axis_name="c", subcore_axis_name="s")

    @functools.partial(
        pl.kernel,
        out_type=jax.ShapeDtypeStruct((NCHUNK * OUT_W,), jnp.float32),
        mesh=mesh,
        compiler_params=pltpu.CompilerParams(
            use_tc_tiling_on_sc=False, needs_layout_passes=False
        ),
        scratch_types=[
            pltpu.VMEM((CPW, CH), jnp.int32),
            pltpu.VMEM((CPW, CH), jnp.int32),
            pltpu.VMEM((CH, 16), jnp.float32),
            pltpu.VMEM((CH, 16), jnp.float32),
            pltpu.VMEM((CH, 16), jnp.float32),
            pltpu.VMEM((CH, 16), jnp.float32),
            pltpu.VMEM((OUT_W,), jnp.float32),
            pltpu.VMEM((OUT_W,), jnp.float32),
            pltpu.SemaphoreType.DMA,
            pltpu.SemaphoreType.DMA,
            pltpu.SemaphoreType.DMA,
            pltpu.SemaphoreType.DMA,
            pltpu.SemaphoreType.DMA,
            pltpu.SemaphoreType.DMA,
        ],
    )
    def k(t_hbm, o_hbm, d_hbm, out_hbm, io, idd, a0, a1, b0, b1,
          o0, o1, sa0, sa1, sb0, sb1, so0, so1):
        wid = lax.axis_index("s") * info.num_cores + lax.axis_index("c")
        base_c = wid * CPW
        is_last = wid == NW - 1
        n = jnp.where(is_last, LASTN, CPW)

        @pl.when(jnp.logical_not(is_last))
        def _():
            pltpu.sync_copy(o_hbm.at[pl.ds(base_c, CPW)], io)
            pltpu.sync_copy(d_hbm.at[pl.ds(base_c, CPW)], idd)

        @pl.when(is_last)
        def _():
            pltpu.sync_copy(
                o_hbm.at[pl.ds(base_c, LASTN)], io.at[pl.ds(0, LASTN)]
            )
            pltpu.sync_copy(
                d_hbm.at[pl.ds(base_c, LASTN)], idd.at[pl.ds(0, LASTN)]
            )

        A = [a0, a1]
        B = [b0, b1]
        O = [o0, o1]
        SA = [sa0, sa1]
        SB = [sb0, sb1]
        SO = [so0, so1]

        iota = lax.iota(jnp.int32, 16)

        def start_gathers(jj, p):
            pltpu.make_async_copy(t_hbm.at[io.at[jj]], A[p], SA[p]).start()
            pltpu.make_async_copy(t_hbm.at[idd.at[jj]], B[p], SB[p]).start()

        def wait_gathers(p):
            pltpu.make_async_copy(t_hbm.at[io.at[0]], A[p], SA[p]).wait()
            pltpu.make_async_copy(t_hbm.at[idd.at[0]], B[p], SB[p]).wait()

        def start_out(jj, p):
            off = pl.multiple_of((base_c + jj) * OUT_W, OUT_W)
            pltpu.make_async_copy(
                O[p], out_hbm.at[pl.ds(off, OUT_W)], SO[p]
            ).start()

        def wait_out(p):
            pltpu.make_async_copy(
                O[p], out_hbm.at[pl.ds(0, OUT_W)], SO[p]
            ).wait()

        def compute(p):
            for h in range(NUM_HEADS):
                ch = jnp.full((16,), h, jnp.int32)
                ch8 = jnp.full((16,), h + 8, jnp.int32)
                for kk in range(CH // 16):
                    ridx = iota + (kk * 16)
                    va = plsc.load_gather(A[p], [ridx, ch])
                    vb = plsc.load_gather(B[p], [ridx, ch8])
                    O[p][pl.ds(h * CH + kk * 16, 16)] = va + vb

        start_gathers(0, 0)
        start_gathers(1, 1)

        def body(t, _):
            j0 = t * 2
            for p in (0, 1):
                jj = j0 + p
                wait_gathers(p)

                @pl.when(t > 0)
                def _():
                    wait_out(p)

                compute(p)
                start_out(jj, p)

                @pl.when(jj + 2 < n)
                def _():
                    start_gathers(jj + 2, p)

            return 0

        lax.fori_loop(0, n // 2, body, 0)
        wait_out(0)
        wait_out(1)

    return k(T, o2, d2)


def kernel(x, edge_index, W, b):
    W2 = jnp.concatenate([W[:D_FEAT], W[D_FEAT:]], axis=1)
    b2 = jnp.concatenate([b, jnp.zeros((NUM_HEADS,), jnp.float32)])
    T = _tc_table(x, W2, b2.reshape(1, 2 * NUM_HEADS))

    ei = edge_index.astype(jnp.int32).reshape(2, NCHUNK, CH)
    out_flat = _sc_edge_combine(T, ei[0], ei[1])
    return (
        out_flat.reshape(NCHUNK, NUM_HEADS, CH)
        .transpose(0, 2, 1)
        .reshape(N_EDGES, NUM_HEADS)
    )
